# Initial kernel scaffold; baseline (speedup 1.0000x reference)
import functools

import jax
import jax.numpy as jnp
from jax import lax
from jax.experimental import pallas as pl
from jax.experimental.pallas import tpu as pltpu
from jax.experimental.pallas import tpu_sc as plsc

N = 10000
E = 320000
D = 128
NC = 2
NS = 16
NW = NC * NS
NPAD = 10240
K = 80
R2D = E // K
ROWS1 = E // K // NS
ROWS2 = E // K // NW


def _prep_body(h_ref, wfc_ref, wa_ref, z_ref, sl_ref, sr_ref):
    h = h_ref[...]
    z = lax.dot_general(h, wfc_ref[...], (((1,), (1,)), ((), ())),
                        preferred_element_type=jnp.float32)
    z_ref[...] = z
    a2 = wa_ref[...].reshape(2, D)
    s2 = lax.dot_general(z, a2, (((1,), (1,)), ((), ())),
                         preferred_element_type=jnp.float32)
    sl_ref[...] = s2[:, 0:1]
    sr_ref[...] = s2[:, 1:2]


def _add_body(a_ref, b_ref, o_ref):
    o_ref[...] = a_ref[...] + b_ref[...]


_sc_mesh = plsc.VectorSubcoreMesh(
    core_axis_name="c", subcore_axis_name="s", num_cores=NC, num_subcores=NS)


@functools.partial(
    pl.kernel,
    out_type=jax.ShapeDtypeStruct((NC, N, D), jnp.float32),
    mesh=_sc_mesh,
    scratch_types=[
        pltpu.VMEM_SHARED((N, D), jnp.float32),
        pltpu.VMEM_SHARED((NPAD,), jnp.float32),
        pltpu.VMEM_SHARED((R2D, K), jnp.float32),
    ],
)
def _sc_gat(src_hbm, dst_hbm, sl_hbm, sr_hbm, z_hbm, out_hbm,
            hacc_sh, den_sh, ex_sh):
    c = lax.axis_index("c")
    s = lax.axis_index("s")
    wid = c * NS + s

    def phase1(sl_v, sr_v, src_v, dst_v, ex_v, zb):
        pltpu.sync_copy(sl_hbm, sl_v)
        pltpu.sync_copy(sr_hbm, sr_v)
        base1 = s * ROWS1
        pltpu.sync_copy(src_hbm.at[pl.ds(base1, ROWS1)], src_v)
        pltpu.sync_copy(dst_hbm.at[pl.ds(base1, ROWS1)], dst_v)

        @pl.loop(0, 40)
        def _zb(i):
            zb[pl.ds(i * 16, 16)] = jnp.zeros((16,), jnp.float32)

        pltpu.sync_copy(zb, den_sh.at[pl.ds(s * 640, 640)])

        def table_max(tab):
            def body(i, acc):
                return jnp.maximum(acc, tab[pl.ds(i * 16, 16)])
            acc = lax.fori_loop(0, N // 16, body,
                                jnp.full((16,), -jnp.inf, jnp.float32))
            return jnp.max(acc)

        cmax = table_max(sl_v) + table_max(sr_v)
        cmax = jnp.maximum(cmax, 0.01 * cmax)
        cv = jnp.broadcast_to(cmax, (16,))

        plsc.subcore_barrier()

        @pl.loop(0, ROWS1)
        def _edge(j):
            for k in range(K // 16):
                sidx = src_v[j, pl.ds(k * 16, 16)]
                didx = dst_v[j, pl.ds(k * 16, 16)]
                x = plsc.load_gather(sl_v, [sidx]) + plsc.load_gather(sr_v, [didx])
                e = jnp.maximum(x, 0.01 * x)
                ex_v[j, pl.ds(k * 16, 16)] = jnp.exp(e - cv)
            pltpu.sync_copy(ex_v.at[j], den_sh.at[dst_v.at[j]], add=True)

        pltpu.sync_copy(ex_v, ex_sh.at[pl.ds(base1, ROWS1)])
        plsc.subcore_barrier()

    pl.run_scoped(
        phase1,
        pltpu.VMEM((N,), jnp.float32),
        pltpu.VMEM((N,), jnp.float32),
        pltpu.VMEM((ROWS1, K), jnp.int32),
        pltpu.VMEM((ROWS1, K), jnp.int32),
        pltpu.VMEM((ROWS1, K), jnp.float32),
        pltpu.VMEM((640,), jnp.float32),
    )

    def phase2(rcp_v, src_v, dst_v, al_v, row0, stage):
        @pl.loop(0, 125)
        def _zr(r):
            for t in range(D // 16):
                stage[r, pl.ds(t * 16, 16)] = jnp.zeros((16,), jnp.float32)

        for q in range(5):
            pltpu.sync_copy(stage, hacc_sh.at[pl.ds(s * 625 + q * 125, 125)])

        pltpu.sync_copy(den_sh, rcp_v)

        @pl.loop(0, NPAD // 16)
        def _rcp(i):
            v = rcp_v[pl.ds(i * 16, 16)]
            rcp_v[pl.ds(i * 16, 16)] = jnp.where(v == 0.0, 1.0, 1.0 / v)

        base2 = wid * ROWS2
        pltpu.sync_copy(src_hbm.at[pl.ds(base2, ROWS2)], src_v)
        pltpu.sync_copy(dst_hbm.at[pl.ds(base2, ROWS2)], dst_v)
        pltpu.sync_copy(ex_sh.at[pl.ds(base2, ROWS2)], al_v)

        plsc.subcore_barrier()

        @pl.loop(0, ROWS2)
        def _chunk(j):
            pltpu.sync_copy(z_hbm.at[src_v.at[j]], row0)
            for k in range(K // 16):
                d16 = dst_v[j, pl.ds(k * 16, 16)]
                r16 = plsc.load_gather(rcp_v, [d16])
                al_v[j, pl.ds(k * 16, 16)] = al_v[j, pl.ds(k * 16, 16)] * r16

            @pl.loop(0, K)
            def _row(r):
                av = jnp.broadcast_to(al_v[j, r], (16,))
                for t in range(D // 16):
                    row0[r, pl.ds(t * 16, 16)] = row0[r, pl.ds(t * 16, 16)] * av

            pltpu.sync_copy(row0, hacc_sh.at[dst_v.at[j]], add=True)

        plsc.subcore_barrier()

        for q in range(5):
            r0 = s * 625 + q * 125
            pltpu.sync_copy(hacc_sh.at[pl.ds(r0, 125)], stage)
            pltpu.sync_copy(stage, out_hbm.at[c].at[pl.ds(r0, 125)])

    pl.run_scoped(
        phase2,
        pltpu.VMEM((NPAD,), jnp.float32),
        pltpu.VMEM((ROWS2, K), jnp.int32),
        pltpu.VMEM((ROWS2, K), jnp.int32),
        pltpu.VMEM((ROWS2, K), jnp.float32),
        pltpu.VMEM((K, D), jnp.float32),
        pltpu.VMEM((125, D), jnp.float32),
    )


def kernel(h, edge_index, W_fc, W_attn):
    src = edge_index[0].astype(jnp.int32).reshape(R2D, K)
    dst = edge_index[1].astype(jnp.int32).reshape(R2D, K)

    z, sl, sr = pl.pallas_call(
        _prep_body,
        grid=(10,),
        in_specs=[
            pl.BlockSpec((N // 10, D), lambda i: (i, 0)),
            pl.BlockSpec((D, D), lambda i: (0, 0)),
            pl.BlockSpec((1, 2 * D), lambda i: (0, 0)),
        ],
        out_specs=[
            pl.BlockSpec((N // 10, D), lambda i: (i, 0)),
            pl.BlockSpec((N // 10, 1), lambda i: (i, 0)),
            pl.BlockSpec((N // 10, 1), lambda i: (i, 0)),
        ],
        out_shape=[
            jax.ShapeDtypeStruct((N, D), jnp.float32),
            jax.ShapeDtypeStruct((N, 1), jnp.float32),
            jax.ShapeDtypeStruct((N, 1), jnp.float32),
        ],
    )(h, W_fc, W_attn)

    hpart = _sc_gat(src, dst, sl.reshape(N), sr.reshape(N), z)

    h_out = pl.pallas_call(
        _add_body,
        grid=(10,),
        in_specs=[
            pl.BlockSpec((N // 10, D), lambda i: (i, 0)),
            pl.BlockSpec((N // 10, D), lambda i: (i, 0)),
        ],
        out_specs=pl.BlockSpec((N // 10, D), lambda i: (i, 0)),
        out_shape=jax.ShapeDtypeStruct((N, D), jnp.float32),
    )(hpart[0], hpart[1])
    return h_out


# speedup vs baseline: 21.3153x; 21.3153x over previous
import functools

import jax
import jax.numpy as jnp
from jax import lax
from jax.experimental import pallas as pl
from jax.experimental.pallas import tpu as pltpu
from jax.experimental.pallas import tpu_sc as plsc

N = 10000
E = 320000
D = 128
NC = 2
NS = 16
NW = NC * NS
NPAD = 10240
K = 80
R2D = E // K
ROWS1 = E // K // NS
ROWS2 = E // K // NW


def _prep_body(h_ref, wfc_ref, wa_ref, z_ref, sl_ref, sr_ref):
    h = h_ref[...]
    z = lax.dot_general(h, wfc_ref[...], (((1,), (1,)), ((), ())),
                        preferred_element_type=jnp.float32)
    z_ref[...] = z
    a2 = wa_ref[...].reshape(2, D)
    s2 = lax.dot_general(z, a2, (((1,), (1,)), ((), ())),
                         preferred_element_type=jnp.float32)
    sl_ref[...] = s2[:, 0:1]
    sr_ref[...] = s2[:, 1:2]


def _add_body(a_ref, b_ref, o_ref):
    o_ref[...] = a_ref[...] + b_ref[...]


_sc_mesh = plsc.VectorSubcoreMesh(
    core_axis_name="c", subcore_axis_name="s", num_cores=NC, num_subcores=NS)


@functools.partial(
    pl.kernel,
    out_type=jax.ShapeDtypeStruct((NC, N, D), jnp.float32),
    mesh=_sc_mesh,
    compiler_params=pltpu.CompilerParams(
        needs_layout_passes=False, use_tc_tiling_on_sc=False),
    scratch_types=[
        pltpu.VMEM_SHARED((N, D), jnp.float32),
        pltpu.VMEM_SHARED((NPAD,), jnp.float32),
    ],
)
def _sc_gat(srcA_hbm, dstA_hbm, sl_hbm, sr_hbm, z_hbm, out_hbm,
            hacc_sh, den_sh):
    c = lax.axis_index("c")
    s = lax.axis_index("s")
    pl.run_scoped(
        functools.partial(_sc_gat_body, srcA_hbm, dstA_hbm, sl_hbm, sr_hbm,
                          z_hbm, out_hbm, hacc_sh, den_sh, c, s),
        pltpu.VMEM((ROWS2, K), jnp.int32),
        pltpu.VMEM((ROWS2, K), jnp.int32),
        pltpu.VMEM((ROWS2, K), jnp.float32),
        pltpu.VMEM((K,), jnp.float32),
        pltpu.VMEM((N,), jnp.float32),
        pltpu.VMEM((K, D), jnp.float32),
    )


def _sc_gat_body(srcA_hbm, dstA_hbm, sl_hbm, sr_hbm, z_hbm, out_hbm,
                 hacc_sh, den_sh, c, s,
                 src_v, dst_v, ex_v, exrow, tabA, buf2d):

    def phase1(sl_v, sr_v):
        pltpu.sync_copy(sl_hbm, sl_v)
        pltpu.sync_copy(sr_hbm, sr_v)

        for k in range(K // 16):
            exrow[pl.ds(k * 16, 16)] = jnp.zeros((16,), jnp.float32)

        @pl.loop(0, 640 // K)
        def _zd(q):
            pltpu.sync_copy(exrow, den_sh.at[pl.ds(s * 640 + q * K, K)])

        def lane_max(acc):
            dnums = lax.GatherDimensionNumbers(
                offset_dims=(), collapsed_slice_dims=(0,), start_index_map=(0,))
            for sh in (8, 4, 2, 1):
                idx = lax.iota(jnp.int32, 16) ^ sh
                perm = lax.gather(
                    acc, idx[:, None], dnums, slice_sizes=(1,),
                    mode=lax.GatherScatterMode.PROMISE_IN_BOUNDS)
                acc = jnp.maximum(acc, perm)
            return acc

        def body_l(i, acc):
            return jnp.maximum(acc, sl_v[pl.ds(i * 16, 16)])

        def body_r(r, acc):
            for t in range(D // 16):
                acc = jnp.maximum(acc, sr_v[r, pl.ds(t * 16, 16)])
            return acc

        neg = jnp.full((16,), -jnp.inf, jnp.float32)
        cv = (lane_max(lax.fori_loop(0, N // 16, body_l, neg))
              + lane_max(lax.fori_loop(0, K, body_r, neg)))
        cv = jnp.maximum(cv, 0.01 * cv)

        plsc.subcore_barrier()

        @pl.loop(0, NC)
        def _half(b):
            pltpu.sync_copy(srcA_hbm.at[s].at[b], src_v)
            pltpu.sync_copy(dstA_hbm.at[s].at[b], dst_v)

            @pl.loop(0, ROWS2)
            def _edge(j):
                for k in range(K // 16):
                    sidx = src_v[j, pl.ds(k * 16, 16)]
                    didx = dst_v[j, pl.ds(k * 16, 16)]
                    x = (plsc.load_gather(sl_v, [sidx])
                         + plsc.load_gather(sr_v, [didx >> 7, didx & 127]))
                    e = jnp.maximum(x, 0.01 * x)
                    exrow[pl.ds(k * 16, 16)] = jnp.exp(e - cv)
                pltpu.sync_copy(exrow, den_sh.at[dst_v.at[j]], add=True)

                @pl.when(b == c)
                def _keep():
                    for k in range(K // 16):
                        ex_v[j, pl.ds(k * 16, 16)] = exrow[pl.ds(k * 16, 16)]

        plsc.subcore_barrier()

    phase1(tabA, buf2d)

    def phase2(rcp_v, row0):
        @pl.loop(0, K)
        def _zr(r):
            for t in range(D // 16):
                row0[r, pl.ds(t * 16, 16)] = jnp.zeros((16,), jnp.float32)

        nzh = jnp.where(s == NS - 1, 5, 8)

        @pl.loop(0, nzh)
        def _zh(q):
            pltpu.sync_copy(row0, hacc_sh.at[pl.ds(s * 640 + q * K, K)])

        pltpu.sync_copy(den_sh.at[pl.ds(0, N)], rcp_v)

        @pl.loop(0, N // 16)
        def _rcp(i):
            v = rcp_v[pl.ds(i * 16, 16)]
            rcp_v[pl.ds(i * 16, 16)] = jnp.where(v == 0.0, 1.0, 1.0 / v)

        pltpu.sync_copy(srcA_hbm.at[s].at[c], src_v)
        pltpu.sync_copy(dstA_hbm.at[s].at[c], dst_v)

        plsc.subcore_barrier()

        @pl.loop(0, ROWS2)
        def _chunk(j):
            pltpu.sync_copy(z_hbm.at[src_v.at[j]], row0)
            for k in range(K // 16):
                d16 = dst_v[j, pl.ds(k * 16, 16)]
                r16 = plsc.load_gather(rcp_v, [d16])
                ex_v[j, pl.ds(k * 16, 16)] = ex_v[j, pl.ds(k * 16, 16)] * r16

            @pl.loop(0, K)
            def _row(r):
                av = plsc.load_gather(
                    ex_v, [jnp.full((16,), j, jnp.int32),
                           jnp.full((16,), r, jnp.int32)])
                for t in range(D // 16):
                    row0[r, pl.ds(t * 16, 16)] = row0[r, pl.ds(t * 16, 16)] * av

            pltpu.sync_copy(row0, hacc_sh.at[dst_v.at[j]], add=True)

        plsc.subcore_barrier()

        nch = jnp.where(s == NS - 1, 5, 8)

        @pl.loop(0, nch)
        def _wb(q):
            r0 = s * 640 + q * K
            pltpu.sync_copy(hacc_sh.at[pl.ds(r0, K)], row0)
            pltpu.sync_copy(row0, out_hbm.at[c].at[pl.ds(r0, K)])

    phase2(tabA, buf2d)


def kernel(h, edge_index, W_fc, W_attn):
    src = edge_index[0].astype(jnp.int32)
    dst = edge_index[1].astype(jnp.int32)
    srcA = src.reshape(NS, NC, ROWS2, K)
    dstA = dst.reshape(NS, NC, ROWS2, K)

    z, sl, sr = pl.pallas_call(
        _prep_body,
        grid=(10,),
        in_specs=[
            pl.BlockSpec((N // 10, D), lambda i: (i, 0)),
            pl.BlockSpec((D, D), lambda i: (0, 0)),
            pl.BlockSpec((1, 2 * D), lambda i: (0, 0)),
        ],
        out_specs=[
            pl.BlockSpec((N // 10, D), lambda i: (i, 0)),
            pl.BlockSpec((N // 10, 1), lambda i: (i, 0)),
            pl.BlockSpec((N // 10, 1), lambda i: (i, 0)),
        ],
        out_shape=[
            jax.ShapeDtypeStruct((N, D), jnp.float32),
            jax.ShapeDtypeStruct((N, 1), jnp.float32),
            jax.ShapeDtypeStruct((N, 1), jnp.float32),
        ],
    )(h, W_fc, W_attn)

    sr_pad = jnp.pad(sr.reshape(N), (0, K * D - N)).reshape(K, D)
    hpart = _sc_gat(srcA, dstA, sl.reshape(N), sr_pad, z)

    h_out = pl.pallas_call(
        _add_body,
        grid=(10,),
        in_specs=[
            pl.BlockSpec((N // 10, D), lambda i: (i, 0)),
            pl.BlockSpec((N // 10, D), lambda i: (i, 0)),
        ],
        out_specs=pl.BlockSpec((N // 10, D), lambda i: (i, 0)),
        out_shape=jax.ShapeDtypeStruct((N, D), jnp.float32),
    )(hpart[0], hpart[1])
    return h_out


# speedup vs baseline: 22.6125x; 1.0609x over previous
import functools

import jax
import jax.numpy as jnp
from jax import lax
from jax.experimental import pallas as pl
from jax.experimental.pallas import tpu as pltpu
from jax.experimental.pallas import tpu_sc as plsc

N = 10000
E = 320000
D = 128
NC = 2
NS = 16
NW = NC * NS
NPAD = 10240
K = 80
R2D = E // K
ROWS1 = E // K // NS
ROWS2 = E // K // NW


def _prep_body(h_ref, wfc_ref, wa_ref, z_ref, sl_ref, sr_ref):
    h = h_ref[...]
    z = lax.dot_general(h, wfc_ref[...], (((1,), (1,)), ((), ())),
                        preferred_element_type=jnp.float32)
    z_ref[...] = z
    a2 = wa_ref[...].reshape(2, D)
    s2 = lax.dot_general(z, a2, (((1,), (1,)), ((), ())),
                         preferred_element_type=jnp.float32)
    sl_ref[...] = s2[:, 0:1]
    sr_ref[...] = s2[:, 1:2]


def _add_body(a_ref, b_ref, o_ref):
    o_ref[...] = a_ref[...] + b_ref[...]


_sc_mesh = plsc.VectorSubcoreMesh(
    core_axis_name="c", subcore_axis_name="s", num_cores=NC, num_subcores=NS)


@functools.partial(
    pl.kernel,
    out_type=jax.ShapeDtypeStruct((NC, N, D), jnp.float32),
    mesh=_sc_mesh,
    compiler_params=pltpu.CompilerParams(
        needs_layout_passes=False, use_tc_tiling_on_sc=False),
    scratch_types=[
        pltpu.VMEM_SHARED((N, D), jnp.float32),
        pltpu.VMEM_SHARED((NPAD,), jnp.float32),
    ],
)
def _sc_gat(srcA_hbm, dstA_hbm, sl_hbm, sr_hbm, z_hbm, out_hbm,
            hacc_sh, den_sh):
    c = lax.axis_index("c")
    s = lax.axis_index("s")
    pl.run_scoped(
        functools.partial(_sc_gat_body, srcA_hbm, dstA_hbm, sl_hbm, sr_hbm,
                          z_hbm, out_hbm, hacc_sh, den_sh, c, s),
        pltpu.VMEM((ROWS2, K), jnp.int32),
        pltpu.VMEM((ROWS2, K), jnp.int32),
        pltpu.VMEM((ROWS2, K), jnp.float32),
        pltpu.VMEM((K,), jnp.float32),
        pltpu.VMEM((N,), jnp.float32),
        pltpu.VMEM((K, D), jnp.float32),
        pltpu.SemaphoreType.DMA,
    )


def _sc_gat_body(srcA_hbm, dstA_hbm, sl_hbm, sr_hbm, z_hbm, out_hbm,
                 hacc_sh, den_sh, c, s,
                 src_v, dst_v, ex_v, exrow, tabA, buf2d, dsem):

    def phase1(sl_v, sr_v):
        pltpu.sync_copy(sl_hbm, sl_v)
        pltpu.sync_copy(sr_hbm, sr_v)

        for k in range(K // 16):
            exrow[pl.ds(k * 16, 16)] = jnp.zeros((16,), jnp.float32)

        @pl.loop(0, 640 // K)
        def _zd(q):
            pltpu.sync_copy(exrow, den_sh.at[pl.ds(s * 640 + q * K, K)])

        def lane_max(acc):
            dnums = lax.GatherDimensionNumbers(
                offset_dims=(), collapsed_slice_dims=(0,), start_index_map=(0,))
            for sh in (8, 4, 2, 1):
                idx = lax.iota(jnp.int32, 16) ^ sh
                perm = lax.gather(
                    acc, idx[:, None], dnums, slice_sizes=(1,),
                    mode=lax.GatherScatterMode.PROMISE_IN_BOUNDS)
                acc = jnp.maximum(acc, perm)
            return acc

        def body_l(i, acc):
            return jnp.maximum(acc, sl_v[pl.ds(i * 16, 16)])

        def body_r(r, acc):
            for t in range(D // 16):
                acc = jnp.maximum(acc, sr_v[r, pl.ds(t * 16, 16)])
            return acc

        neg = jnp.full((16,), -jnp.inf, jnp.float32)
        cv = (lane_max(lax.fori_loop(0, N // 16, body_l, neg))
              + lane_max(lax.fori_loop(0, K, body_r, neg)))
        cv = jnp.maximum(cv, 0.01 * cv)

        plsc.subcore_barrier()

        @pl.loop(0, NC)
        def _half(q):
            b = 1 - c + q * (2 * c - 1)
            pltpu.sync_copy(srcA_hbm.at[s].at[b], src_v)
            pltpu.sync_copy(dstA_hbm.at[s].at[b], dst_v)

            @pl.loop(0, ROWS2)
            def _edge(j):
                for k in range(K // 16):
                    sidx = src_v[j, pl.ds(k * 16, 16)]
                    didx = dst_v[j, pl.ds(k * 16, 16)]
                    x = (plsc.load_gather(sl_v, [sidx])
                         + plsc.load_gather(sr_v, [didx >> 7, didx & 127]))
                    e = jnp.maximum(x, 0.01 * x)
                    ex_v[j, pl.ds(k * 16, 16)] = jnp.exp(e - cv)
                pltpu.async_copy(ex_v.at[j], den_sh.at[dst_v.at[j]], dsem,
                                 add=True)

            @pl.loop(0, ROWS2)
            def _dr(j):
                pltpu.make_async_copy(
                    ex_v.at[0], den_sh.at[dst_v.at[0]], dsem).wait()

        plsc.subcore_barrier()

    phase1(tabA, buf2d)

    def phase2(rcp_v, row0):
        @pl.loop(0, K)
        def _zr(r):
            for t in range(D // 16):
                row0[r, pl.ds(t * 16, 16)] = jnp.zeros((16,), jnp.float32)

        nzh = jnp.where(s == NS - 1, 5, 8)

        @pl.loop(0, nzh)
        def _zh(q):
            pltpu.sync_copy(row0, hacc_sh.at[pl.ds(s * 640 + q * K, K)])

        pltpu.sync_copy(den_sh.at[pl.ds(0, N)], rcp_v)

        @pl.loop(0, N // 16)
        def _rcp(i):
            v = rcp_v[pl.ds(i * 16, 16)]
            rcp_v[pl.ds(i * 16, 16)] = jnp.where(v == 0.0, 1.0, 1.0 / v)

        pltpu.sync_copy(srcA_hbm.at[s].at[c], src_v)
        pltpu.sync_copy(dstA_hbm.at[s].at[c], dst_v)

        plsc.subcore_barrier()

        @pl.loop(0, ROWS2)
        def _chunk(j):
            pltpu.sync_copy(z_hbm.at[src_v.at[j]], row0)
            for k in range(K // 16):
                d16 = dst_v[j, pl.ds(k * 16, 16)]
                r16 = plsc.load_gather(rcp_v, [d16])
                ex_v[j, pl.ds(k * 16, 16)] = ex_v[j, pl.ds(k * 16, 16)] * r16

            @pl.loop(0, K)
            def _row(r):
                av = plsc.load_gather(
                    ex_v, [jnp.full((16,), j, jnp.int32),
                           jnp.full((16,), r, jnp.int32)])
                for t in range(D // 16):
                    row0[r, pl.ds(t * 16, 16)] = row0[r, pl.ds(t * 16, 16)] * av

            pltpu.sync_copy(row0, hacc_sh.at[dst_v.at[j]], add=True)

        plsc.subcore_barrier()

        nch = jnp.where(s == NS - 1, 5, 8)

        @pl.loop(0, nch)
        def _wb(q):
            r0 = s * 640 + q * K
            pltpu.sync_copy(hacc_sh.at[pl.ds(r0, K)], row0)
            pltpu.sync_copy(row0, out_hbm.at[c].at[pl.ds(r0, K)])

    phase2(tabA, buf2d)


def kernel(h, edge_index, W_fc, W_attn):
    src = edge_index[0].astype(jnp.int32)
    dst = edge_index[1].astype(jnp.int32)
    srcA = src.reshape(NS, NC, ROWS2, K)
    dstA = dst.reshape(NS, NC, ROWS2, K)

    z, sl, sr = pl.pallas_call(
        _prep_body,
        grid=(10,),
        in_specs=[
            pl.BlockSpec((N // 10, D), lambda i: (i, 0)),
            pl.BlockSpec((D, D), lambda i: (0, 0)),
            pl.BlockSpec((1, 2 * D), lambda i: (0, 0)),
        ],
        out_specs=[
            pl.BlockSpec((N // 10, D), lambda i: (i, 0)),
            pl.BlockSpec((N // 10, 1), lambda i: (i, 0)),
            pl.BlockSpec((N // 10, 1), lambda i: (i, 0)),
        ],
        out_shape=[
            jax.ShapeDtypeStruct((N, D), jnp.float32),
            jax.ShapeDtypeStruct((N, 1), jnp.float32),
            jax.ShapeDtypeStruct((N, 1), jnp.float32),
        ],
    )(h, W_fc, W_attn)

    sr_pad = jnp.pad(sr.reshape(N), (0, K * D - N)).reshape(K, D)
    hpart = _sc_gat(srcA, dstA, sl.reshape(N), sr_pad, z)

    h_out = pl.pallas_call(
        _add_body,
        grid=(10,),
        in_specs=[
            pl.BlockSpec((N // 10, D), lambda i: (i, 0)),
            pl.BlockSpec((N // 10, D), lambda i: (i, 0)),
        ],
        out_specs=pl.BlockSpec((N // 10, D), lambda i: (i, 0)),
        out_shape=jax.ShapeDtypeStruct((N, D), jnp.float32),
    )(hpart[0], hpart[1])
    return h_out
